# Initial kernel scaffold; baseline (speedup 1.0000x reference)
import jax
import jax.numpy as jnp
from jax.experimental import pallas as pl


def kernel(speaker_ids, id2_index, table):
    raise NotImplementedError("write your pallas kernel here")


# speedup vs baseline: 1.9952x; 1.9952x over previous
import jax
import jax.numpy as jnp
from jax import lax
from jax.experimental import pallas as pl
from jax.experimental.pallas import tpu as pltpu
from jax.experimental.pallas import tpu_sc as plsc

NUM_SPEAKERS = 520
HIDDEN_DIM = 1024
BATCH = 16384

_INFO = plsc.get_sparse_core_info()
_NC, _NS, _L = _INFO.num_cores, _INFO.num_subcores, _INFO.num_lanes
_NW = _NC * _NS
_BPW = BATCH // _NW
_CHUNK = 32
_NCHUNK = _BPW // _CHUNK
_NBUF = 3


def _body(ids_hbm, id2_hbm, table_hbm, out_hbm,
          ids_v, idx_v, rows_v, rsem, in_sems, out_sems):
    wid = lax.axis_index("s") * _NC + lax.axis_index("c")
    base = wid * _BPW

    pltpu.sync_copy(ids_hbm.at[pl.ds(base, _BPW)], ids_v)
    remap_h = [
        pltpu.async_copy(id2_hbm.at[ids_v.at[pl.ds(c * _CHUNK, _CHUNK)]],
                         idx_v.at[c], rsem)
        for c in range(_NCHUNK)
    ]
    for h in remap_h:
        h.wait()

    def gather_start(c, buf):
        return pltpu.async_copy(table_hbm.at[idx_v.at[c]], rows_v.at[buf],
                                in_sems.at[buf])

    def out_start(c, buf):
        return pltpu.async_copy(rows_v.at[buf],
                                out_hbm.at[pl.ds(base + c * _CHUNK, _CHUNK)],
                                out_sems.at[buf])

    in_h = [None] * _NBUF
    out_h = [None] * _NBUF
    for b in range(min(_NBUF, _NCHUNK)):
        in_h[b] = gather_start(b, b)
    for c in range(_NCHUNK):
        buf = c % _NBUF
        in_h[buf].wait()
        out_h[buf] = out_start(c, buf)
        nxt = c + _NBUF
        if nxt < _NCHUNK:
            out_h[buf].wait()
            out_h[buf] = None
            in_h[buf] = gather_start(nxt, buf)
    for b in range(_NBUF):
        if out_h[b] is not None:
            out_h[b].wait()


@jax.jit
def _lookup(speaker_ids, id2_index, table):
    mesh = plsc.VectorSubcoreMesh(core_axis_name="c", subcore_axis_name="s")
    ker = pl.kernel(
        _body,
        out_type=jax.ShapeDtypeStruct((BATCH, HIDDEN_DIM), jnp.float32),
        mesh=mesh,
        scratch_types=[
            pltpu.VMEM((_BPW,), jnp.int32),
            pltpu.VMEM((_NCHUNK, _CHUNK), jnp.int32),
            pltpu.VMEM((_NBUF, _CHUNK, HIDDEN_DIM), jnp.float32),
            pltpu.SemaphoreType.DMA,
            pltpu.SemaphoreType.DMA((_NBUF,)),
            pltpu.SemaphoreType.DMA((_NBUF,)),
        ],
    )
    return ker(speaker_ids.astype(jnp.int32), id2_index.astype(jnp.int32),
               table)


def kernel(speaker_ids, id2_index, table):
    out = _lookup(speaker_ids, id2_index, table)
    return out[:, None, :]


# speedup vs baseline: 3.0456x; 1.5265x over previous
import jax
import jax.numpy as jnp
from jax import lax
from jax.experimental import pallas as pl
from jax.experimental.pallas import tpu as pltpu
from jax.experimental.pallas import tpu_sc as plsc

NUM_SPEAKERS = 520
HIDDEN_DIM = 1024
BATCH = 16384

_INFO = plsc.get_sparse_core_info()
_NC, _NS, _L = _INFO.num_cores, _INFO.num_subcores, _INFO.num_lanes
_NW = _NC * _NS
_BPW = BATCH // _NW
_CHUNK = 32
_NCHUNK = _BPW // _CHUNK
_NBUF = 3


def _body(ids_hbm, id2_hbm, table_hbm, out_hbm,
          ids_v, idx_v, rows_v, rsem, in_sems, out_sems):
    wid = lax.axis_index("s") * _NC + lax.axis_index("c")
    base = wid * _BPW

    pltpu.sync_copy(ids_hbm.at[pl.ds(base, _BPW)], ids_v)
    remap_h = [
        pltpu.async_copy(id2_hbm.at[ids_v.at[pl.ds(c * _CHUNK, _CHUNK)]],
                         idx_v.at[c], rsem)
        for c in range(_NCHUNK)
    ]
    for h in remap_h:
        h.wait()

    def gather_start(c, buf):
        return pltpu.async_copy(table_hbm.at[idx_v.at[c]], rows_v.at[buf],
                                in_sems.at[buf])

    def out_start(c, buf):
        return pltpu.async_copy(rows_v.at[buf],
                                out_hbm.at[pl.ds(base + c * _CHUNK, _CHUNK), 0],
                                out_sems.at[buf])

    in_h = [None] * _NBUF
    out_h = [None] * _NBUF
    for b in range(min(_NBUF, _NCHUNK)):
        in_h[b] = gather_start(b, b)
    for c in range(_NCHUNK):
        buf = c % _NBUF
        in_h[buf].wait()
        out_h[buf] = out_start(c, buf)
        nxt = c + _NBUF
        if nxt < _NCHUNK:
            out_h[buf].wait()
            out_h[buf] = None
            in_h[buf] = gather_start(nxt, buf)
    for b in range(_NBUF):
        if out_h[b] is not None:
            out_h[b].wait()


@jax.jit
def _lookup(speaker_ids, id2_index, table):
    mesh = plsc.VectorSubcoreMesh(core_axis_name="c", subcore_axis_name="s")
    ker = pl.kernel(
        _body,
        out_type=jax.ShapeDtypeStruct((BATCH, 1, HIDDEN_DIM), jnp.float32),
        mesh=mesh,
        scratch_types=[
            pltpu.VMEM((_BPW,), jnp.int32),
            pltpu.VMEM((_NCHUNK, _CHUNK), jnp.int32),
            pltpu.VMEM((_NBUF, _CHUNK, HIDDEN_DIM), jnp.float32),
            pltpu.SemaphoreType.DMA,
            pltpu.SemaphoreType.DMA((_NBUF,)),
            pltpu.SemaphoreType.DMA((_NBUF,)),
        ],
    )
    return ker(speaker_ids.astype(jnp.int32), id2_index.astype(jnp.int32),
               table)


def kernel(speaker_ids, id2_index, table):
    return _lookup(speaker_ids, id2_index, table)
